# baseline (speedup 1.0000x reference)
import functools

import jax
import jax.numpy as jnp
from jax import lax
from jax.experimental import pallas as pl
from jax.experimental.pallas import tpu as pltpu
from jax.experimental.pallas import tpu_sc as plsc

F32 = jnp.float32
NB = 16
DIM = 256
KC = 512
KH = 128
TT = 784
TCTX = 768
FW = 16
NC, NS = 2, 16
NW = NC * NS
ZQ_W = (NB * TT) // NW
GI_W = (NB * TCTX) // NW
GI_CH = 96


def _enc_body(x_ref, emb_ref, sel_ref, w1_ref, b1_ref, w2_ref, b2_ref, cb_ref,
              zt_ref, idx_ref, tt_ref, hc_ref):
    x2 = jnp.dot(x_ref[0, 0], emb_ref[0], preferred_element_type=F32)
    x2 = x2 + jnp.dot(x_ref[0, 1], emb_ref[1], preferred_element_type=F32)
    x2 = x2 + jnp.dot(x_ref[0, 2], emb_ref[2], preferred_element_type=F32)
    y0 = jnp.dot(sel_ref[0], x2, preferred_element_type=F32)
    y1 = jnp.dot(sel_ref[1], x2, preferred_element_type=F32)
    z0 = y0.reshape(57, 57, 6)
    z1 = y1.reshape(57, 57, 6)
    parts = []
    for dh in (0, 1):
        for dw in (0, 1):
            for zz in (z0, z1):
                parts.append(zz[dh:dh + 56, dw:dw + 56, :])
    a = jnp.concatenate(parts, axis=-1).reshape(3136, 48)
    hm = jnp.maximum(
        jnp.dot(a, w1_ref[...], preferred_element_type=F32) + b1_ref[0][None, :],
        0.0)

    h4 = hm.reshape(28, 2, 56, 256)
    he = h4[:, 0].reshape(28, 28, 2, 256)
    ho = h4[:, 1].reshape(28, 28, 2, 256)
    hc_ref[...] = jnp.zeros((29, 29, 4 * DIM), F32)
    hc_ref[1:29, 1:29, 0:256] = ho[:, :, 1, :]
    hc_ref[1:29, 0:28, 256:512] = ho[:, :, 0, :]
    hc_ref[0:28, 1:29, 512:768] = he[:, :, 1, :]
    hc_ref[0:28, 0:28, 768:1024] = he[:, :, 0, :]
    hc = hc_ref[...]

    w2 = w2_ref[...]
    z = jnp.dot(hc[0:28, 0:28, :].reshape(784, 1024), w2[0],
                preferred_element_type=F32)
    z = z + jnp.dot(hc[0:28, 1:29, :].reshape(784, 1024), w2[1],
                    preferred_element_type=F32)
    z = z + jnp.dot(hc[1:29, 0:28, :].reshape(784, 1024), w2[2],
                    preferred_element_type=F32)
    z = z + jnp.dot(hc[1:29, 1:29, :].reshape(784, 1024), w2[3],
                    preferred_element_type=F32)
    z = z + b2_ref[0][None, :]

    cb = cb_ref[...]
    zsq = jnp.sum(z * z, axis=-1, keepdims=True)
    csq = jnp.sum(cb * cb, axis=-1)
    s = lax.dot_general(z, cb, (((1,), (1,)), ((), ())),
                        preferred_element_type=F32)
    d2 = zsq - 2.0 * s + csq[None, :]
    m = jnp.min(d2, axis=-1, keepdims=True)
    io = lax.broadcasted_iota(jnp.int32, (784, KC), 1)
    idx = jnp.min(jnp.where(d2 <= m, io, KC), axis=-1)
    zt_ref[...] = z
    idx_ref[0, 0] = idx
    tt_ref[0] = z[TCTX:TT, :]


def _gitab_body(cb_ref, wih_ref, bih_ref, o_ref):
    o_ref[...] = (jnp.dot(cb_ref[...], wih_ref[...], preferred_element_type=F32)
                  + bih_ref[0][None, :])


@functools.lru_cache(maxsize=1)
def _make_sc_gathers():
    mesh = plsc.VectorSubcoreMesh(core_axis_name="c", subcore_axis_name="s")

    @functools.partial(
        pl.kernel,
        out_type=jax.ShapeDtypeStruct((NB * TCTX, 3 * KH), F32),
        mesh=mesh,
        scratch_types=[
            pltpu.VMEM((GI_W // GI_CH, GI_CH), jnp.int32),
            pltpu.VMEM((GI_CH, 3 * KH), F32),
            pltpu.VMEM((GI_CH, 3 * KH), F32),
            pltpu.VMEM((GI_CH, 3 * KH), F32),
            pltpu.SemaphoreType.DMA,
            pltpu.SemaphoreType.DMA,
            pltpu.SemaphoreType.DMA,
            pltpu.SemaphoreType.DMA,
        ],
    )
    def gi_gather(gtab_hbm, idxc_hbm, gi_out, idxc_v, r0, r1, r2,
                  s0, s1, s2, so):
        wid = lax.axis_index("s") * NC + lax.axis_index("c")
        pltpu.sync_copy(idxc_hbm.at[wid], idxc_v)
        base = wid * GI_W
        bufs = (r0, r1, r2)
        sems = (s0, s1, s2)
        nch = GI_W // GI_CH
        cps = [pltpu.async_copy(gtab_hbm.at[idxc_v.at[j]], bufs[j], sems[j])
               for j in range(3)]
        outs = []
        drained = 0
        for j in range(nch):
            cps[j].wait()
            outs.append(pltpu.async_copy(
                bufs[j % 3], gi_out.at[pl.ds(base + j * GI_CH, GI_CH)], so))
            if j + 3 < nch:
                outs[j].wait()
                drained = j + 1
                cps.append(pltpu.async_copy(
                    gtab_hbm.at[idxc_v.at[j + 3]], bufs[j % 3], sems[j % 3]))
        for j in range(drained, nch):
            outs[j].wait()

    @functools.partial(
        pl.kernel,
        out_type=jax.ShapeDtypeStruct((NB * TT, DIM), F32),
        mesh=mesh,
        scratch_types=[
            pltpu.VMEM((ZQ_W,), jnp.int32),
            pltpu.VMEM((ZQ_W, DIM), F32),
            pltpu.SemaphoreType.DMA,
        ],
    )
    def zq_gather(cb_hbm, idxa_hbm, zq_out, idxa_v, rows, sem):
        wid = lax.axis_index("s") * NC + lax.axis_index("c")
        pltpu.sync_copy(idxa_hbm.at[wid], idxa_v)
        base = wid * ZQ_W
        pltpu.async_copy(cb_hbm.at[idxa_v], rows, sem).wait()
        pltpu.sync_copy(rows, zq_out.at[pl.ds(base, ZQ_W)])

    return gi_gather, zq_gather


def _gru_body(gi_ref, whh_ref, h0_ref, wp_ref, tt_ref, acc_ref, nce_ref):
    whh_r = whh_ref[:, 0:KH]
    whh_z = whh_ref[:, KH:2 * KH]
    whh_n = whh_ref[:, 2 * KH:3 * KH]

    def step(t, h):
        gx = gi_ref[t]
        ghr = jnp.dot(h, whh_r, preferred_element_type=F32)
        ghn = jnp.dot(h, whh_n, preferred_element_type=F32)
        ghz = jnp.dot(h, whh_z, preferred_element_type=F32)
        r = jax.nn.sigmoid(gx[:, 0:KH] + ghr)
        n = jnp.tanh(gx[:, 2 * KH:3 * KH] + r * ghn)
        zg = jax.nn.sigmoid(gx[:, KH:2 * KH] + ghz)
        return (1.0 - zg) * n + zg * h

    h = lax.fori_loop(0, TCTX, step, h0_ref[...], unroll=2)

    tt = tt_ref[...]
    io = lax.broadcasted_iota(jnp.int32, (NB, NB), 1)
    lab = lax.broadcasted_iota(jnp.int32, (NB, NB), 0)
    eye = io == lab
    nce_sum = F32(0.0)
    acc_sum = F32(0.0)
    for k in range(FW):
        pred = jnp.dot(h, wp_ref[k], preferred_element_type=F32)
        tg = tt[:, k, :]
        sc = lax.dot_general(pred, tg, (((1,), (1,)), ((), ())),
                             preferred_element_type=F32)
        m = jnp.max(sc, axis=-1, keepdims=True)
        lse = m + jnp.log(jnp.sum(jnp.exp(sc - m), axis=-1, keepdims=True))
        diag = jnp.sum(jnp.where(eye, sc, 0.0), axis=-1, keepdims=True)
        nce_sum = nce_sum + jnp.sum(diag - lse)
        am = jnp.min(jnp.where(sc >= m, io, NB), axis=-1, keepdims=True)
        lab1 = lax.broadcasted_iota(jnp.int32, (NB, 1), 0)
        acc_sum = acc_sum + jnp.sum((am == lab1).astype(F32))
    nce_ref[...] = jnp.reshape(-nce_sum / F32(FW * NB), (1, 1))
    acc_ref[...] = jnp.reshape(acc_sum / F32(FW * NB), (1, 1))


def kernel(x, hidden, conv1_w, conv1_b, conv2_w, conv2_b, codebook,
           W_ih, W_hh, b_ih, b_hh, W_pred):
    w1 = (conv1_w.transpose(2, 3, 1, 0).reshape(2, 2, 2, 2, 3, DIM)
          .transpose(0, 2, 1, 3, 4, 5).reshape(48, DIM))
    w2 = (conv2_w.transpose(2, 3, 1, 0).reshape(2, 2, 2, 2, DIM, DIM)
          .transpose(0, 2, 1, 3, 4, 5).reshape(4, 4 * DIM, DIM))
    wsrc = jnp.arange(112)
    lane = jnp.arange(342)
    emb = jnp.stack([
        (lane[None, :] == 3 * (wsrc[:, None] + 1) + i).astype(F32)
        for i in range(3)])
    hp = jnp.arange(57)
    rows = jnp.arange(112)
    sel = jnp.stack([
        (rows[None, :] == 2 * hp[:, None] - 1).astype(F32),
        (rows[None, :] == 2 * hp[:, None]).astype(F32)])

    zt, idx3, tt = pl.pallas_call(
        _enc_body,
        grid=(NB,),
        in_specs=[
            pl.BlockSpec((1, 3, 112, 112), lambda b: (b, 0, 0, 0)),
            pl.BlockSpec((3, 112, 342), lambda b: (0, 0, 0)),
            pl.BlockSpec((2, 57, 112), lambda b: (0, 0, 0)),
            pl.BlockSpec((48, DIM), lambda b: (0, 0)),
            pl.BlockSpec((1, DIM), lambda b: (0, 0)),
            pl.BlockSpec((4, 4 * DIM, DIM), lambda b: (0, 0, 0)),
            pl.BlockSpec((1, DIM), lambda b: (0, 0)),
            pl.BlockSpec((KC, DIM), lambda b: (0, 0)),
        ],
        out_specs=[
            pl.BlockSpec((TT, DIM), lambda b: (0, b)),
            pl.BlockSpec((1, 1, TT), lambda b: (b, 0, 0)),
            pl.BlockSpec((1, FW, DIM), lambda b: (b, 0, 0)),
        ],
        out_shape=[
            jax.ShapeDtypeStruct((TT, NB * DIM), F32),
            jax.ShapeDtypeStruct((NB, 1, TT), jnp.int32),
            jax.ShapeDtypeStruct((NB, FW, DIM), F32),
        ],
        scratch_shapes=[pltpu.VMEM((29, 29, 4 * DIM), F32)],
    )(x, emb, sel, w1, conv1_b.reshape(1, DIM), w2, conv2_b.reshape(1, DIM),
      codebook)

    gi_tab = pl.pallas_call(
        _gitab_body,
        in_specs=[
            pl.BlockSpec((KC, DIM), lambda: (0, 0)),
            pl.BlockSpec((DIM, 3 * KH), lambda: (0, 0)),
            pl.BlockSpec((1, 3 * KH), lambda: (0, 0)),
        ],
        out_specs=pl.BlockSpec((KC, 3 * KH), lambda: (0, 0)),
        out_shape=jax.ShapeDtypeStruct((KC, 3 * KH), F32),
    )(codebook, W_ih.T, (b_ih + b_hh).reshape(1, 3 * KH))

    idx = idx3.reshape(NB, TT)
    idx_t = idx.T
    idx_all = idx_t.reshape(NW, ZQ_W)
    idx_ctx = idx_t[:TCTX].reshape(NW, GI_W // GI_CH, GI_CH)

    gi_gather, zq_gather = _make_sc_gathers()
    gi = gi_gather(gi_tab, idx_ctx)
    z_q_flat = zq_gather(codebook, idx_all)

    acc2, nce2 = pl.pallas_call(
        _gru_body,
        in_specs=[
            pl.BlockSpec((TCTX, NB, 3 * KH), lambda: (0, 0, 0)),
            pl.BlockSpec((KH, 3 * KH), lambda: (0, 0)),
            pl.BlockSpec((NB, KH), lambda: (0, 0)),
            pl.BlockSpec((FW, KH, DIM), lambda: (0, 0, 0)),
            pl.BlockSpec((NB, FW, DIM), lambda: (0, 0, 0)),
        ],
        out_specs=[
            pl.BlockSpec((1, 1), lambda: (0, 0)),
            pl.BlockSpec((1, 1), lambda: (0, 0)),
        ],
        out_shape=[
            jax.ShapeDtypeStruct((1, 1), F32),
            jax.ShapeDtypeStruct((1, 1), F32),
        ],
    )(gi.reshape(TCTX, NB, 3 * KH), W_hh.T, hidden[0], W_pred, tt)

    z_e_x = jnp.transpose(zt.reshape(28, 28, NB, DIM), (2, 3, 0, 1))
    z_q_x = jnp.transpose(z_q_flat.reshape(28, 28, NB, DIM), (2, 3, 0, 1))
    return acc2[0, 0], nce2[0, 0], z_e_x, z_q_x


# speedup vs baseline: 1.0272x; 1.0272x over previous
import functools

import jax
import jax.numpy as jnp
from jax import lax
from jax.experimental import pallas as pl
from jax.experimental.pallas import tpu as pltpu
from jax.experimental.pallas import tpu_sc as plsc

F32 = jnp.float32
NB = 16
DIM = 256
KC = 512
KH = 128
TT = 784
TCTX = 768
FW = 16
NC, NS = 2, 16
NW = NC * NS
ZQ_W = (NB * TT) // NW
GI_W = (NB * TCTX) // NW
GI_CH = 96


def _enc_body(x_ref, emb_ref, sel_ref, w1_ref, b1_ref, w2_ref, b2_ref, cb_ref,
              zt_ref, idx_ref, tt_ref, hc_ref):
    x2 = jnp.dot(x_ref[0, 0], emb_ref[0], preferred_element_type=F32)
    x2 = x2 + jnp.dot(x_ref[0, 1], emb_ref[1], preferred_element_type=F32)
    x2 = x2 + jnp.dot(x_ref[0, 2], emb_ref[2], preferred_element_type=F32)
    y0 = jnp.dot(sel_ref[0], x2, preferred_element_type=F32)
    y1 = jnp.dot(sel_ref[1], x2, preferred_element_type=F32)
    z0 = y0.reshape(57, 57, 6)
    z1 = y1.reshape(57, 57, 6)
    parts = []
    for dh in (0, 1):
        for dw in (0, 1):
            for zz in (z0, z1):
                parts.append(zz[dh:dh + 56, dw:dw + 56, :])
    a = jnp.concatenate(parts, axis=-1).reshape(3136, 48)
    hm = jnp.maximum(
        jnp.dot(a, w1_ref[...], preferred_element_type=F32) + b1_ref[0][None, :],
        0.0)

    h4 = hm.reshape(28, 2, 56, 256)
    he = h4[:, 0].reshape(28, 28, 2, 256)
    ho = h4[:, 1].reshape(28, 28, 2, 256)
    hc_ref[...] = jnp.zeros((29, 29, 4 * DIM), F32)
    hc_ref[1:29, 1:29, 0:256] = ho[:, :, 1, :]
    hc_ref[1:29, 0:28, 256:512] = ho[:, :, 0, :]
    hc_ref[0:28, 1:29, 512:768] = he[:, :, 1, :]
    hc_ref[0:28, 0:28, 768:1024] = he[:, :, 0, :]
    hc = hc_ref[...]

    w2 = w2_ref[...]
    z = jnp.dot(hc[0:28, 0:28, :].reshape(784, 1024), w2[0],
                preferred_element_type=F32)
    z = z + jnp.dot(hc[0:28, 1:29, :].reshape(784, 1024), w2[1],
                    preferred_element_type=F32)
    z = z + jnp.dot(hc[1:29, 0:28, :].reshape(784, 1024), w2[2],
                    preferred_element_type=F32)
    z = z + jnp.dot(hc[1:29, 1:29, :].reshape(784, 1024), w2[3],
                    preferred_element_type=F32)
    z = z + b2_ref[0][None, :]

    cb = cb_ref[...]
    zsq = jnp.sum(z * z, axis=-1, keepdims=True)
    csq = jnp.sum(cb * cb, axis=-1)
    s = lax.dot_general(z, cb, (((1,), (1,)), ((), ())),
                        preferred_element_type=F32)
    d2 = zsq - 2.0 * s + csq[None, :]
    m = jnp.min(d2, axis=-1, keepdims=True)
    io = lax.broadcasted_iota(jnp.int32, (784, KC), 1)
    idx = jnp.min(jnp.where(d2 <= m, io, KC), axis=-1)
    zt_ref[...] = z
    idx_ref[0, 0] = idx
    tt_ref[0] = z[TCTX:TT, :]


def _gitab_body(cb_ref, wih_ref, bih_ref, o_ref):
    o_ref[...] = (jnp.dot(cb_ref[...], wih_ref[...], preferred_element_type=F32)
                  + bih_ref[0][None, :])


@functools.lru_cache(maxsize=1)
def _make_sc_gathers():
    mesh = plsc.VectorSubcoreMesh(core_axis_name="c", subcore_axis_name="s")

    @functools.partial(
        pl.kernel,
        out_type=jax.ShapeDtypeStruct((NB * TCTX, 3 * KH), F32),
        mesh=mesh,
        scratch_types=[
            pltpu.VMEM((GI_W // GI_CH, GI_CH), jnp.int32),
            pltpu.VMEM((GI_CH, 3 * KH), F32),
            pltpu.VMEM((GI_CH, 3 * KH), F32),
            pltpu.VMEM((GI_CH, 3 * KH), F32),
            pltpu.SemaphoreType.DMA,
            pltpu.SemaphoreType.DMA,
            pltpu.SemaphoreType.DMA,
            pltpu.SemaphoreType.DMA,
        ],
    )
    def gi_gather(gtab_hbm, idxc_hbm, gi_out, idxc_v, r0, r1, r2,
                  s0, s1, s2, so):
        wid = lax.axis_index("s") * NC + lax.axis_index("c")
        pltpu.sync_copy(idxc_hbm.at[wid], idxc_v)
        base = wid * GI_W
        bufs = (r0, r1, r2)
        sems = (s0, s1, s2)
        nch = GI_W // GI_CH
        cps = [pltpu.async_copy(gtab_hbm.at[idxc_v.at[j]], bufs[j], sems[j])
               for j in range(3)]
        outs = []
        drained = 0
        for j in range(nch):
            cps[j].wait()
            outs.append(pltpu.async_copy(
                bufs[j % 3], gi_out.at[pl.ds(base + j * GI_CH, GI_CH)], so))
            if j + 3 < nch:
                outs[j].wait()
                drained = j + 1
                cps.append(pltpu.async_copy(
                    gtab_hbm.at[idxc_v.at[j + 3]], bufs[j % 3], sems[j % 3]))
        for j in range(drained, nch):
            outs[j].wait()

    @functools.partial(
        pl.kernel,
        out_type=jax.ShapeDtypeStruct((NB * TT, DIM), F32),
        mesh=mesh,
        scratch_types=[
            pltpu.VMEM((ZQ_W,), jnp.int32),
            pltpu.VMEM((ZQ_W, DIM), F32),
            pltpu.SemaphoreType.DMA,
        ],
    )
    def zq_gather(cb_hbm, idxa_hbm, zq_out, idxa_v, rows, sem):
        wid = lax.axis_index("s") * NC + lax.axis_index("c")
        pltpu.sync_copy(idxa_hbm.at[wid], idxa_v)
        base = wid * ZQ_W
        pltpu.async_copy(cb_hbm.at[idxa_v], rows, sem).wait()
        pltpu.sync_copy(rows, zq_out.at[pl.ds(base, ZQ_W)])

    return gi_gather, zq_gather


def _gru_body(gi_ref, whh_ref, h0_ref, wp_ref, tt_ref, acc_ref, nce_ref):
    whh = whh_ref[...]

    def substep(gx, h):
        gh = jnp.dot(h, whh, preferred_element_type=F32)
        r = jax.nn.sigmoid(gx[:, 0:KH] + gh[:, 0:KH])
        zg = jax.nn.sigmoid(gx[:, KH:2 * KH] + gh[:, KH:2 * KH])
        n = jnp.tanh(gx[:, 2 * KH:3 * KH] + r * gh[:, 2 * KH:3 * KH])
        return (1.0 - zg) * n + zg * h

    def step(t, hs):
        ha, hb = hs
        gx = gi_ref[t]
        return substep(gx[0:8], ha), substep(gx[8:16], hb)

    h0 = h0_ref[...]
    ha, hb = lax.fori_loop(0, TCTX, step, (h0[0:8], h0[8:16]), unroll=4)
    h = jnp.concatenate([ha, hb], axis=0)

    tt = tt_ref[...]
    io = lax.broadcasted_iota(jnp.int32, (NB, NB), 1)
    lab = lax.broadcasted_iota(jnp.int32, (NB, NB), 0)
    eye = io == lab
    nce_sum = F32(0.0)
    acc_sum = F32(0.0)
    for k in range(FW):
        pred = jnp.dot(h, wp_ref[k], preferred_element_type=F32)
        tg = tt[:, k, :]
        sc = lax.dot_general(pred, tg, (((1,), (1,)), ((), ())),
                             preferred_element_type=F32)
        m = jnp.max(sc, axis=-1, keepdims=True)
        lse = m + jnp.log(jnp.sum(jnp.exp(sc - m), axis=-1, keepdims=True))
        diag = jnp.sum(jnp.where(eye, sc, 0.0), axis=-1, keepdims=True)
        nce_sum = nce_sum + jnp.sum(diag - lse)
        am = jnp.min(jnp.where(sc >= m, io, NB), axis=-1, keepdims=True)
        lab1 = lax.broadcasted_iota(jnp.int32, (NB, 1), 0)
        acc_sum = acc_sum + jnp.sum((am == lab1).astype(F32))
    nce_ref[...] = jnp.reshape(-nce_sum / F32(FW * NB), (1, 1))
    acc_ref[...] = jnp.reshape(acc_sum / F32(FW * NB), (1, 1))


def kernel(x, hidden, conv1_w, conv1_b, conv2_w, conv2_b, codebook,
           W_ih, W_hh, b_ih, b_hh, W_pred):
    w1 = (conv1_w.transpose(2, 3, 1, 0).reshape(2, 2, 2, 2, 3, DIM)
          .transpose(0, 2, 1, 3, 4, 5).reshape(48, DIM))
    w2 = (conv2_w.transpose(2, 3, 1, 0).reshape(2, 2, 2, 2, DIM, DIM)
          .transpose(0, 2, 1, 3, 4, 5).reshape(4, 4 * DIM, DIM))
    wsrc = jnp.arange(112)
    lane = jnp.arange(342)
    emb = jnp.stack([
        (lane[None, :] == 3 * (wsrc[:, None] + 1) + i).astype(F32)
        for i in range(3)])
    hp = jnp.arange(57)
    rows = jnp.arange(112)
    sel = jnp.stack([
        (rows[None, :] == 2 * hp[:, None] - 1).astype(F32),
        (rows[None, :] == 2 * hp[:, None]).astype(F32)])

    zt, idx3, tt = pl.pallas_call(
        _enc_body,
        grid=(NB,),
        in_specs=[
            pl.BlockSpec((1, 3, 112, 112), lambda b: (b, 0, 0, 0)),
            pl.BlockSpec((3, 112, 342), lambda b: (0, 0, 0)),
            pl.BlockSpec((2, 57, 112), lambda b: (0, 0, 0)),
            pl.BlockSpec((48, DIM), lambda b: (0, 0)),
            pl.BlockSpec((1, DIM), lambda b: (0, 0)),
            pl.BlockSpec((4, 4 * DIM, DIM), lambda b: (0, 0, 0)),
            pl.BlockSpec((1, DIM), lambda b: (0, 0)),
            pl.BlockSpec((KC, DIM), lambda b: (0, 0)),
        ],
        out_specs=[
            pl.BlockSpec((TT, DIM), lambda b: (0, b)),
            pl.BlockSpec((1, 1, TT), lambda b: (b, 0, 0)),
            pl.BlockSpec((1, FW, DIM), lambda b: (b, 0, 0)),
        ],
        out_shape=[
            jax.ShapeDtypeStruct((TT, NB * DIM), F32),
            jax.ShapeDtypeStruct((NB, 1, TT), jnp.int32),
            jax.ShapeDtypeStruct((NB, FW, DIM), F32),
        ],
        scratch_shapes=[pltpu.VMEM((29, 29, 4 * DIM), F32)],
    )(x, emb, sel, w1, conv1_b.reshape(1, DIM), w2, conv2_b.reshape(1, DIM),
      codebook)

    gi_tab = pl.pallas_call(
        _gitab_body,
        in_specs=[
            pl.BlockSpec((KC, DIM), lambda: (0, 0)),
            pl.BlockSpec((DIM, 3 * KH), lambda: (0, 0)),
            pl.BlockSpec((1, 3 * KH), lambda: (0, 0)),
        ],
        out_specs=pl.BlockSpec((KC, 3 * KH), lambda: (0, 0)),
        out_shape=jax.ShapeDtypeStruct((KC, 3 * KH), F32),
    )(codebook, W_ih.T, (b_ih + b_hh).reshape(1, 3 * KH))

    idx = idx3.reshape(NB, TT)
    idx_t = idx.T
    idx_all = idx_t.reshape(NW, ZQ_W)
    idx_ctx = idx_t[:TCTX].reshape(NW, GI_W // GI_CH, GI_CH)

    gi_gather, zq_gather = _make_sc_gathers()
    gi = gi_gather(gi_tab, idx_ctx)
    z_q_flat = zq_gather(codebook, idx_all)

    acc2, nce2 = pl.pallas_call(
        _gru_body,
        in_specs=[
            pl.BlockSpec((TCTX, NB, 3 * KH), lambda: (0, 0, 0)),
            pl.BlockSpec((KH, 3 * KH), lambda: (0, 0)),
            pl.BlockSpec((NB, KH), lambda: (0, 0)),
            pl.BlockSpec((FW, KH, DIM), lambda: (0, 0, 0)),
            pl.BlockSpec((NB, FW, DIM), lambda: (0, 0, 0)),
        ],
        out_specs=[
            pl.BlockSpec((1, 1), lambda: (0, 0)),
            pl.BlockSpec((1, 1), lambda: (0, 0)),
        ],
        out_shape=[
            jax.ShapeDtypeStruct((1, 1), F32),
            jax.ShapeDtypeStruct((1, 1), F32),
        ],
    )(gi.reshape(TCTX, NB, 3 * KH), W_hh.T, hidden[0], W_pred, tt)

    z_e_x = jnp.transpose(zt.reshape(28, 28, NB, DIM), (2, 3, 0, 1))
    z_q_x = jnp.transpose(z_q_flat.reshape(28, 28, NB, DIM), (2, 3, 0, 1))
    return acc2[0, 0], nce2[0, 0], z_e_x, z_q_x
